# baseline (speedup 1.0000x reference)
import functools

import jax
import jax.numpy as jnp
from jax import lax
from jax.experimental import pallas as pl
from jax.experimental.pallas import tpu as pltpu
from jax.experimental.pallas import tpu_sc as plsc

N = 10000
NP = 10240
E = 320000
HID = 256
HEADS = 8
DH = 128
NSUB = 16
EPT = E // NSUB
K = 80
NCHUNK = EPT // K
ZR = 8
F32 = jnp.float32

_mesh = functools.partial(
    plsc.VectorSubcoreMesh, core_axis_name="c", subcore_axis_name="s")

_GDN = lax.GatherDimensionNumbers(
    offset_dims=(), collapsed_slice_dims=(0,), start_index_map=(0,))


def _lane_gather(x, idx):
    return lax.gather(x, idx[:, None], _GDN, (1,),
                      mode=lax.GatherScatterMode.PROMISE_IN_BOUNDS)


_K2 = 200
_LW = NP * 8


def _loopattr_body(eapf, dst, out, idxb, ebf, lab):
    c = lax.axis_index("c")
    s = lax.axis_index("s")
    lane = lax.iota(jnp.int32, 16)
    lane8 = lane & 7
    mask8 = lane < 8

    @pl.loop(0, _LW // DH)
    def _zero(r):
        for j in range(DH // 16):
            lab[r, pl.ds(j * 16, 16)] = jnp.zeros((16,), F32)

    half = E // 2

    @pl.loop(0, half // (NSUB * _K2))
    def _chunk(i):
        base = c * half + s * (half // NSUB) + i * _K2
        pltpu.sync_copy(dst.at[pl.ds(base, _K2)], idxb)
        pltpu.sync_copy(eapf.at[pl.ds(base * 8, _K2 * 8)],
                        ebf.at[pl.ds(0, _K2 * 8)])
        for q in range(_K2 // 16):
            dch = idxb[pl.ds(q * 16, 16)]
            for e2 in range(16):
                dv = _lane_gather(dch, jnp.broadcast_to(e2, (16,)).astype(jnp.int32))
                vals = ebf[pl.ds((q * 16 + e2) * 8, 16)]
                w = dv * 8 + lane8
                plsc.addupdate_scatter(
                    lab, [lax.shift_right_logical(w, 7), w & 127], vals,
                    mask=mask8)

    pltpu.sync_copy(lab, out.at[pl.ds((c * NSUB + s) * (_LW // DH), _LW // DH)])


@functools.cache
def _loopattr_call():
    return pl.kernel(
        _loopattr_body,
        out_type=jax.ShapeDtypeStruct((2 * NSUB * (_LW // DH), DH), F32),
        mesh=_mesh(),
        compiler_params=pltpu.CompilerParams(needs_layout_passes=False),
        scratch_types=[
            pltpu.VMEM((_K2,), jnp.int32),
            pltpu.VMEM((_K2 * 8 + 8,), F32),
            pltpu.VMEM((_LW // DH, DH), F32),
        ],
    )


_DW = NP * 4
_AR = 5008
_NH = N // 2
_TR = 320
_T15 = _AR - 15 * _TR


def _edge_body(src2, dst, dst2, ee4, xl2, xr2, att4,
               msg_out, den_out,
               idx_s, idx_d, idx_d0, idx_p, xlb, xrb, eeb, ctb, zb,
               attb, denb, accum,
               sem_s, sem_d, sem_m, sem_ee, sem_g1, sem_g2, sem_sc):
    c = lax.axis_index("c")
    s = lax.axis_index("s")

    @pl.loop(0, _DW // DH)
    def _zero_d(r):
        for j in range(DH // 16):
            denb[r, pl.ds(j * 16, 16)] = jnp.zeros((16,), F32)

    @pl.loop(0, ZR)
    def _zero(r):
        for j in range(DH // 16):
            zb[r, pl.ds(j * 16, 16)] = jnp.zeros((16,), F32)

    nz = _TR // ZR
    nz15 = _T15 // ZR
    lane = lax.iota(jnp.int32, 16)
    perms = [lane ^ k for k in (1, 2, 4, 8)]
    lane2 = lane & 1
    mask2 = lane < 2

    for q in range(2):
        p = 2 * c + q
        pltpu.sync_copy(att4.at[pl.ds(p * 128, 128)], attb)
        attj = [attb[pl.ds(j * 16, 16)] for j in range(4)]

        for kk in range(nz):
            @pl.when(jnp.logical_or(s < 15, kk < nz15))
            def _():
                pltpu.sync_copy(zb, accum.at[pl.ds(s * _TR + kk * ZR, ZR)])
        plsc.subcore_barrier()

        def base_of(chunk):
            cb = jnp.minimum(chunk, NCHUNK - 1)
            return s * EPT + cb * K

        def issue_sd(chunk, bs):
            b = base_of(chunk)
            pltpu.async_copy(src2.at[pl.ds(c * E + b, K)], idx_s[bs], sem_s[bs])
            pltpu.async_copy(dst2.at[pl.ds(c * E + b, K)], idx_d[bs], sem_d[bs])

        def issue_d0sp(chunk, bs):
            b = base_of(chunk)
            pltpu.async_copy(dst.at[pl.ds(b, K)], idx_d0[bs], sem_m[bs])

        def wait_sd(bs):
            pltpu.make_async_copy(src2.at[pl.ds(0, K)], idx_s[bs], sem_s[bs]).wait()
            pltpu.make_async_copy(dst2.at[pl.ds(0, K)], idx_d[bs], sem_d[bs]).wait()

        def wait_d0sp(bs):
            pltpu.make_async_copy(dst.at[pl.ds(0, K)], idx_d0[bs], sem_m[bs]).wait()

        def drain_sc():
            pltpu.make_async_copy(ctb, accum.at[idx_p], sem_sc).wait()

        def compute(bs):
            for ii in range(K // 16):
                dv16 = idx_d0[bs][pl.ds(ii * 16, 16)]
                idx_p[pl.ds(ii * 16, 16)] = lax.shift_right_logical(dv16, 1)
            for s16 in range(K // 16):
                dch = idx_d0[bs][pl.ds(s16 * 16, 16)]

                @plsc.parallel_loop(0, 16, unroll=4)
                def _edge(e16):
                    e = s16 * 16 + e16
                    e16v = jnp.broadcast_to(e16, (16,)).astype(jnp.int32)
                    de = _lane_gather(dch, e16v)
                    dhi = (de & 1) > 0
                    xlj = []
                    pj = []
                    for j in range(4):
                        xv = xlb[e, pl.ds(q * 64 + j * 16, 16)]
                        rv = xrb[e, pl.ds(q * 64 + j * 16, 16)]
                        m = xv + rv + eeb[e, pl.ds(j * 16, 16)]
                        m = jnp.where(m > 0, m, m * F32(0.2))
                        xlj.append(xv)
                        pj.append(m * attj[j])
                    dv = jnp.zeros((16,), F32)
                    z = jnp.zeros((16,), F32)
                    for h in range(2):
                        a = pj[2 * h] + pj[2 * h + 1]
                        for pidx in perms:
                            a = a + _lane_gather(a, pidx)
                        wv = jnp.exp(a)
                        v0 = wv * xlj[2 * h]
                        v1 = wv * xlj[2 * h + 1]
                        ctb[e, pl.ds(32 * h, 16)] = jnp.where(dhi, z, v0)
                        ctb[e, pl.ds(32 * h + 16, 16)] = jnp.where(dhi, z, v1)
                        ctb[e, pl.ds(64 + 32 * h, 16)] = jnp.where(dhi, v0, z)
                        ctb[e, pl.ds(64 + 32 * h + 16, 16)] = jnp.where(
                            dhi, v1, z)
                        dv = dv + jnp.where(lane == h, wv, F32(0.0))
                    w = de * 4 + (2 * q + lane2)
                    plsc.addupdate_scatter(
                        denb, [lax.shift_right_logical(w, 7), w & 127], dv,
                        mask=mask2)

        issue_sd(0, 0)
        issue_d0sp(0, 0)

        @pl.loop(0, NCHUNK // 2)
        def _chunk(i2):
            for par in (0, 1):
                i = 2 * i2 + par
                other = 1 - par
                wait_sd(par)
                pltpu.async_copy(xl2.at[idx_s[par]], xlb, sem_g1)
                pltpu.async_copy(xr2.at[idx_d[par]], xrb, sem_g2)
                pltpu.async_copy(ee4.at[pl.ds(p * E + base_of(i), K)], eeb,
                                 sem_ee)
                if par == 0:
                    @pl.when(i2 > 0)
                    def _():
                        drain_sc()
                else:
                    drain_sc()
                issue_sd(i + 1, other)
                issue_d0sp(i + 1, other)
                wait_d0sp(par)
                pltpu.make_async_copy(ee4.at[pl.ds(0, K)], eeb, sem_ee).wait()
                pltpu.make_async_copy(xl2.at[idx_s[par]], xlb, sem_g1).wait()
                pltpu.make_async_copy(xr2.at[idx_d[par]], xrb, sem_g2).wait()
                compute(par)
                pltpu.async_copy(ctb, accum.at[idx_p], sem_sc, add=True)

        drain_sc()
        wait_sd(0)
        wait_d0sp(0)

        plsc.subcore_barrier()
        for kk in range(nz):
            @pl.when(jnp.logical_or(s < 15, kk < nz15))
            def _():
                r0 = s * _TR + kk * ZR
                pltpu.sync_copy(accum.at[pl.ds(r0, ZR)], zb)
                pltpu.sync_copy(zb, msg_out.at[pl.ds(p * _AR + r0, ZR)])
        if q == 0:
            @pl.loop(0, ZR)
            def _rezero(r):
                for j in range(DH // 16):
                    zb[r, pl.ds(j * 16, 16)] = jnp.zeros((16,), F32)
        plsc.subcore_barrier()

    pltpu.sync_copy(
        denb, den_out.at[pl.ds((c * NSUB + s) * (_DW // DH), _DW // DH)])


@functools.cache
def _edge_call():
    return pl.kernel(
        _edge_body,
        out_type=[jax.ShapeDtypeStruct((4 * _AR, DH), F32),
                  jax.ShapeDtypeStruct((2 * NSUB * (_DW // DH), DH), F32)],
        mesh=_mesh(),
        compiler_params=pltpu.CompilerParams(
            needs_layout_passes=False, internal_scratch_in_bytes=0),
        scratch_types=[
            [pltpu.VMEM((K,), jnp.int32)] * 2,
            [pltpu.VMEM((K,), jnp.int32)] * 2,
            [pltpu.VMEM((K,), jnp.int32)] * 2,
            pltpu.VMEM((K,), jnp.int32),
            pltpu.VMEM((K, DH), F32),
            pltpu.VMEM((K, DH), F32),
            pltpu.VMEM((K, 64), F32),
            pltpu.VMEM((K, DH), F32),
            pltpu.VMEM((ZR, DH), F32),
            pltpu.VMEM((DH,), F32),
            pltpu.VMEM((_DW // DH, DH), F32),
            pltpu.VMEM_SHARED((_AR, DH), F32),
            [pltpu.SemaphoreType.DMA] * 2,
            [pltpu.SemaphoreType.DMA] * 2,
            [pltpu.SemaphoreType.DMA] * 2,
            pltpu.SemaphoreType.DMA,
            pltpu.SemaphoreType.DMA,
            pltpu.SemaphoreType.DMA,
            pltpu.SemaphoreType.DMA,
        ],
    )


def _ln(x, g, b):
    m = jnp.mean(x, -1, keepdims=True)
    v = jnp.var(x, -1, keepdims=True)
    return (x - m) / jnp.sqrt(v + 1e-5) * g + b


_BN = 2000
_BE = 4000


def _fullb(shape):
    return pl.BlockSpec(shape, lambda i: (0,) * len(shape))


def _enc_body(x_ref, w1, b1, g1r, bl1, w2, b2, o_ref):
    h = jnp.dot(x_ref[...], w1[...], preferred_element_type=F32) + b1[...]
    h = jnp.maximum(h, 0.0)
    h = _ln(h, g1r[...], bl1[...])
    o_ref[...] = jnp.dot(h, w2[...], preferred_element_type=F32) + b2[...]


@functools.cache
def _enc_call():
    return pl.pallas_call(
        _enc_body,
        grid=(N // _BN,),
        in_specs=[pl.BlockSpec((_BN, 128), lambda i: (i, 0)),
                  _fullb((128, 256)), _fullb((256,)), _fullb((256,)),
                  _fullb((256,)), _fullb((256, 128)), _fullb((128,))],
        out_specs=pl.BlockSpec((_BN, 128), lambda i: (i, 0)),
        out_shape=jax.ShapeDtypeStruct((N, 128), F32),
    )


def _pack_halves(x, o_ref):
    for cc in range(2):
        o_ref[cc, :, :] = x[:, 128 * cc:128 * cc + 128]


def _pre_body(h_ref, la_ref, wl_r, blr, wr_r, brr, we_r, att_r, wres_r, bcr,
              xl4_ref, xr4_ref, res_ref, snum_ref, wl_ref):
    h = h_ref[...]
    xl = jnp.dot(h, wl_r[...], preferred_element_type=F32) + blr[...]
    xr = jnp.dot(h, wr_r[...], preferred_element_type=F32) + brr[...]
    res_ref[...] = (jnp.dot(h, wres_r[...], preferred_element_type=F32)
                    + bcr[...])
    _pack_halves(xl, xl4_ref)
    _pack_halves(xr, xr4_ref)
    ml = xl + xr + jnp.dot(la_ref[...], we_r[...], preferred_element_type=F32)
    ml = jnp.where(ml > 0, ml, 0.2 * ml)
    al = jnp.sum(ml.reshape(_BN, 8, 32) * att_r[...][None], -1)
    wl_ = jnp.exp(al)
    wl_ref[...] = wl_
    snum_ref[...] = jnp.repeat(wl_, 32, axis=1) * xl


@functools.cache
def _pre_call(din):
    return pl.pallas_call(
        _pre_body,
        grid=(N // _BN,),
        in_specs=[pl.BlockSpec((_BN, din), lambda i: (i, 0)),
                  pl.BlockSpec((_BN, 8), lambda i: (i, 0)),
                  _fullb((din, 256)), _fullb((256,)),
                  _fullb((din, 256)), _fullb((256,)),
                  _fullb((8, 256)), _fullb((8, 32)),
                  _fullb((din, 256)), _fullb((256,))],
        out_specs=[pl.BlockSpec((2, _BN, 128), lambda i: (0, i, 0)),
                   pl.BlockSpec((2, _BN, 128), lambda i: (0, i, 0)),
                   pl.BlockSpec((_BN, 256), lambda i: (i, 0)),
                   pl.BlockSpec((_BN, 256), lambda i: (i, 0)),
                   pl.BlockSpec((_BN, 8), lambda i: (i, 0))],
        out_shape=[jax.ShapeDtypeStruct((2, N, 128), F32),
                   jax.ShapeDtypeStruct((2, N, 128), F32),
                   jax.ShapeDtypeStruct((N, 256), F32),
                   jax.ShapeDtypeStruct((N, 256), F32),
                   jax.ShapeDtypeStruct((N, 8), F32)],
    )


def _ee_body(ea_ref, we_r, o_ref):
    ee = jnp.dot(ea_ref[...], we_r[...], preferred_element_type=F32)
    for p in range(4):
        o_ref[p, :, :] = ee[:, 64 * p:64 * p + 64]


@functools.cache
def _ee_call():
    return pl.pallas_call(
        _ee_body,
        grid=(E // _BE,),
        in_specs=[pl.BlockSpec((_BE, 8), lambda i: (i, 0)),
                  _fullb((8, 256))],
        out_specs=pl.BlockSpec((4, _BE, 64), lambda i: (0, i, 0)),
        out_shape=jax.ShapeDtypeStruct((4, E, 64), F32),
    )


def _post_body(m4_ref, dp_ref, snum_ref, wl_ref, res_ref, g_r, b_r, o_ref):
    m4 = m4_ref[...]
    parts = ([m4[p, :, 0:64] for p in range(4)]
             + [m4[p, :, 64:128] for p in range(4)])
    msg = jnp.concatenate(parts, axis=-1).reshape(_BN, 256)
    den8 = dp_ref[...] + wl_ref[...]
    num = msg + snum_ref[...]
    out = num / jnp.repeat(den8 + 1e-16, 32, axis=1) + res_ref[...]
    out = _ln(out, g_r[...], b_r[...])
    o_ref[...] = out * jax.nn.sigmoid(out)


@functools.cache
def _post_call():
    return pl.pallas_call(
        _post_body,
        grid=(N // _BN,),
        in_specs=[pl.BlockSpec((4, _BN // 2, 128), lambda i: (0, i, 0)),
                  pl.BlockSpec((_BN, 8), lambda i: (i, 0)),
                  pl.BlockSpec((_BN, 256), lambda i: (i, 0)),
                  pl.BlockSpec((_BN, 8), lambda i: (i, 0)),
                  pl.BlockSpec((_BN, 256), lambda i: (i, 0)),
                  _fullb((256,)), _fullb((256,))],
        out_specs=pl.BlockSpec((_BN, 256), lambda i: (i, 0)),
        out_shape=jax.ShapeDtypeStruct((N, 256), F32),
    )


def _pool_body(h_ref, idx_ref, wq, bq_r, wk, bk_r, wv, bv_r, wo, bo_r,
               o_ref, s_acc, c_acc):
    i = pl.program_id(0)

    @pl.when(i == 0)
    def _():
        s_acc[...] = jnp.zeros_like(s_acc)
        c_acc[...] = jnp.zeros_like(c_acc)

    seg = jax.lax.broadcasted_iota(jnp.int32, (160, _BN), 0)
    oh = (seg == idx_ref[...][0]).astype(F32)
    s_acc[...] += jnp.dot(oh, h_ref[...], preferred_element_type=F32)
    c_acc[...] += jnp.sum(oh, axis=1, keepdims=True)

    @pl.when(i == N // _BN - 1)
    def _():
        dense = s_acc[...] / jnp.maximum(c_acc[...], 1.0)
        q0 = dense.reshape(40, 4, 256)[:, 0, :]
        q = jnp.dot(q0, wq[...], preferred_element_type=F32) + bq_r[...]
        kk = jnp.dot(dense, wk[...], preferred_element_type=F32) + bk_r[...]
        vv = jnp.dot(dense, wv[...], preferred_element_type=F32) + bv_r[...]
        q4 = q.reshape(40, 1, 8, 32)
        k4 = kk.reshape(40, 4, 8, 32)
        v4 = vv.reshape(40, 4, 8, 32)
        logits = jnp.sum(q4 * k4, -1) / jnp.sqrt(F32(32.0))
        mxl = jnp.max(logits, axis=1, keepdims=True)
        ex = jnp.exp(logits - mxl)
        aw = ex / jnp.sum(ex, axis=1, keepdims=True)
        o = jnp.sum(aw[..., None] * v4, axis=1).reshape(40, 256)
        o_ref[...] = jnp.dot(o, wo[...], preferred_element_type=F32) + bo_r[...]


@functools.cache
def _pool_call():
    return pl.pallas_call(
        _pool_body,
        grid=(N // _BN,),
        in_specs=[pl.BlockSpec((_BN, 256), lambda i: (i, 0)),
                  pl.BlockSpec((1, 1, _BN), lambda i: (i, 0, 0)),
                  _fullb((256, 256)), _fullb((256,)),
                  _fullb((256, 256)), _fullb((256,)),
                  _fullb((256, 256)), _fullb((256,)),
                  _fullb((256, 256)), _fullb((256,))],
        out_specs=pl.BlockSpec((40, 256), lambda i: (0, 0)),
        out_shape=jax.ShapeDtypeStruct((40, 256), F32),
        scratch_shapes=[pltpu.VMEM((160, 256), F32),
                        pltpu.VMEM((160, 1), F32)],
    )


def kernel(x, edge_index, edge_attr, batch, t, node_type, node_temporal_mask,
           enc_W1, enc_b1, enc_g, enc_bln, enc_W2, enc_b2,
           Wl0, bl0, Wr0, br0, We0, att0, Wres0, bc0, g0, bn0,
           Wl1, bl1, Wr1, br1, We1, att1, Wres1, bc1, g1, bn1,
           Wl2, bl2, Wr2, br2, We2, att2, Wres2, bc2, g2, bn2,
           Wq, Wk, Wv, Wo, bq, bk, bv, bo):
    src = edge_index[0].astype(jnp.int32)
    dst = edge_index[1].astype(jnp.int32)
    off2 = jnp.repeat(jnp.arange(2, dtype=jnp.int32) * N, E)
    src2 = jnp.tile(src, 2) + off2
    dst2 = jnp.tile(dst, 2) + off2

    eapf = jnp.concatenate(
        [edge_attr, jnp.ones((E, 1), F32), jnp.zeros((E, 1), F32)],
        axis=1).reshape(-1)
    lsum2 = _loopattr_call()(eapf, dst)
    lsum = lsum2.reshape(2 * NSUB, NP, 8).sum(axis=0)[:N]
    cnt = jnp.maximum(lsum[:, 6], 1.0)
    la8 = jnp.concatenate(
        [lsum[:, :6] / cnt[:, None], jnp.zeros((N, 2), F32)], axis=1)

    h = _enc_call()(x, enc_W1, enc_b1, enc_g, enc_bln, enc_W2, enc_b2)
    eap8 = jnp.pad(edge_attr, ((0, 0), (0, 2)))

    layers = [
        (Wl0, bl0, Wr0, br0, We0, att0, Wres0, bc0, g0, bn0),
        (Wl1, bl1, Wr1, br1, We1, att1, Wres1, bc1, g1, bn1),
        (Wl2, bl2, Wr2, br2, We2, att2, Wres2, bc2, g2, bn2),
    ]
    for (Wl, bl, Wr, br, We, att, Wres, bc, g, bn) in layers:
        din = Wl.shape[0]
        We8 = jnp.pad(We, ((0, 2), (0, 0)))
        xl4, xr4, res, snum, wl_ = _pre_call(din)(
            h, la8, Wl, bl, Wr, br, We8, att, Wres, bc)
        ee4 = _ee_call()(eap8, We8).reshape(4 * E, 64)
        att4 = jnp.pad(att.reshape(4, 64), ((0, 0), (0, 64))).reshape(-1)
        msg4, denp = _edge_call()(
            src2, dst, dst2, ee4, xl4.reshape(2 * N, 128),
            xr4.reshape(2 * N, 128), att4)
        m4 = msg4.reshape(4, _AR, 128)[:, :N // 2]
        denh = denp.reshape(2, NSUB, NP, 4).sum(axis=1)
        den8 = jnp.concatenate([denh[0, :N], denh[1, :N]], axis=1)
        h = _post_call()(m4, den8, snum, wl_, res, g, bn)

    Tv = node_temporal_mask.shape[1]
    Bv = node_type.shape[0]
    ntf = jnp.broadcast_to(node_type[:, None, :],
                           (Bv, Tv, node_type.shape[1])).reshape(-1)
    tmf = node_temporal_mask.reshape(-1)
    pool_idx = (batch * (Tv * 4) + tmf * 4 + ntf).astype(jnp.int32)
    o = _pool_call()(h, pool_idx.reshape(N // _BN, 1, _BN),
                     Wq, bq, Wk, bk, Wv, bv, Wo, bo)
    return o.reshape(Bv, Tv, HID)


# speedup vs baseline: 1.1741x; 1.1741x over previous
import functools

import jax
import jax.numpy as jnp
from jax import lax
from jax.experimental import pallas as pl
from jax.experimental.pallas import tpu as pltpu
from jax.experimental.pallas import tpu_sc as plsc

N = 10000
NP = 10240
E = 320000
HID = 256
HEADS = 8
DH = 128
NSUB = 16
EPT = E // NSUB
K = 80
NCHUNK = EPT // K
ZR = 8
F32 = jnp.float32

_mesh = functools.partial(
    plsc.VectorSubcoreMesh, core_axis_name="c", subcore_axis_name="s")

_GDN = lax.GatherDimensionNumbers(
    offset_dims=(), collapsed_slice_dims=(0,), start_index_map=(0,))


def _lane_gather(x, idx):
    return lax.gather(x, idx[:, None], _GDN, (1,),
                      mode=lax.GatherScatterMode.PROMISE_IN_BOUNDS)


_K2 = 200
_LW = NP * 8


def _loopattr_body(eapf, dst, out, idxb, ebf, lab):
    c = lax.axis_index("c")
    s = lax.axis_index("s")
    lane = lax.iota(jnp.int32, 16)
    lane8 = lane & 7
    mask8 = lane < 8

    @pl.loop(0, _LW // DH)
    def _zero(r):
        for j in range(DH // 16):
            lab[r, pl.ds(j * 16, 16)] = jnp.zeros((16,), F32)

    half = E // 2

    @pl.loop(0, half // (NSUB * _K2))
    def _chunk(i):
        base = c * half + s * (half // NSUB) + i * _K2
        pltpu.sync_copy(dst.at[pl.ds(base, _K2)], idxb)
        pltpu.sync_copy(eapf.at[pl.ds(base * 8, _K2 * 8)],
                        ebf.at[pl.ds(0, _K2 * 8)])
        for q in range(_K2 // 16):
            dch = idxb[pl.ds(q * 16, 16)]
            for e2 in range(16):
                dv = _lane_gather(dch, jnp.broadcast_to(e2, (16,)).astype(jnp.int32))
                vals = ebf[pl.ds((q * 16 + e2) * 8, 16)]
                w = dv * 8 + lane8
                plsc.addupdate_scatter(
                    lab, [lax.shift_right_logical(w, 7), w & 127], vals,
                    mask=mask8)

    pltpu.sync_copy(lab, out.at[pl.ds((c * NSUB + s) * (_LW // DH), _LW // DH)])


@functools.cache
def _loopattr_call():
    return pl.kernel(
        _loopattr_body,
        out_type=jax.ShapeDtypeStruct((2 * NSUB * (_LW // DH), DH), F32),
        mesh=_mesh(),
        compiler_params=pltpu.CompilerParams(needs_layout_passes=False),
        scratch_types=[
            pltpu.VMEM((_K2,), jnp.int32),
            pltpu.VMEM((_K2 * 8 + 8,), F32),
            pltpu.VMEM((_LW // DH, DH), F32),
        ],
    )


_DW = NP * 4
_AR = 5008
_NH = N // 2
_TR = 320
_T15 = _AR - 15 * _TR


def _edge_body(src2, dst, dst2, ee4, xl2, xr2, att4,
               msg_out, den_out,
               idx_s, idx_d, idx_d0, idx_p, xlb, xrb, eeb, ctb, zb,
               attb, denb, accum,
               sem_s, sem_d, sem_m, sem_ee, sem_g1, sem_g2, sem_sc):
    c = lax.axis_index("c")
    s = lax.axis_index("s")

    @pl.loop(0, _DW // DH)
    def _zero_d(r):
        for j in range(DH // 16):
            denb[r, pl.ds(j * 16, 16)] = jnp.zeros((16,), F32)

    @pl.loop(0, ZR)
    def _zero(r):
        for j in range(DH // 16):
            zb[r, pl.ds(j * 16, 16)] = jnp.zeros((16,), F32)

    nz = _TR // ZR
    nz15 = _T15 // ZR
    lane = lax.iota(jnp.int32, 16)
    perms = [lane ^ k for k in (1, 2, 4, 8)]
    lane2 = lane & 1
    mask2 = lane < 2

    for q in range(2):
        p = 2 * c + q
        pltpu.sync_copy(att4.at[pl.ds(p * 128, 128)], attb)
        attj = [attb[pl.ds(j * 16, 16)] for j in range(4)]

        for kk in range(nz):
            @pl.when(jnp.logical_or(s < 15, kk < nz15))
            def _():
                pltpu.sync_copy(zb, accum.at[pl.ds(s * _TR + kk * ZR, ZR)])
        plsc.subcore_barrier()

        def base_of(chunk):
            cb = jnp.minimum(chunk, NCHUNK - 1)
            return s * EPT + cb * K

        def issue_sd(chunk, bs):
            b = base_of(chunk)
            pltpu.async_copy(src2.at[pl.ds(c * E + b, K)], idx_s[bs], sem_s[bs])
            pltpu.async_copy(dst2.at[pl.ds(c * E + b, K)], idx_d[bs], sem_d[bs])

        def issue_d0sp(chunk, bs):
            b = base_of(chunk)
            pltpu.async_copy(dst.at[pl.ds(b, K)], idx_d0[bs], sem_m[bs])

        def wait_sd(bs):
            pltpu.make_async_copy(src2.at[pl.ds(0, K)], idx_s[bs], sem_s[bs]).wait()
            pltpu.make_async_copy(dst2.at[pl.ds(0, K)], idx_d[bs], sem_d[bs]).wait()

        def wait_d0sp(bs):
            pltpu.make_async_copy(dst.at[pl.ds(0, K)], idx_d0[bs], sem_m[bs]).wait()

        def drain_sc():
            pltpu.make_async_copy(ctb, accum.at[idx_p], sem_sc).wait()

        def compute(bs):
            for ii in range(K // 16):
                dv16 = idx_d0[bs][pl.ds(ii * 16, 16)]
                idx_p[pl.ds(ii * 16, 16)] = lax.shift_right_logical(dv16, 1)
            for s16 in range(K // 16):
                dch = idx_d0[bs][pl.ds(s16 * 16, 16)]

                @plsc.parallel_loop(0, 16, unroll=2)
                def _edge(e16):
                    e = s16 * 16 + e16
                    e16v = jnp.broadcast_to(e16, (16,)).astype(jnp.int32)
                    de = _lane_gather(dch, e16v)
                    dhi = (de & 1) > 0
                    xlj = []
                    pj = []
                    for j in range(4):
                        xv = xlb[e, pl.ds(q * 64 + j * 16, 16)]
                        rv = xrb[e, pl.ds(q * 64 + j * 16, 16)]
                        m = xv + rv + eeb[e, pl.ds(j * 16, 16)]
                        m = jnp.where(m > 0, m, m * F32(0.2))
                        xlj.append(xv)
                        pj.append(m * attj[j])
                    dv = jnp.zeros((16,), F32)
                    z = jnp.zeros((16,), F32)
                    for h in range(2):
                        a = pj[2 * h] + pj[2 * h + 1]
                        for pidx in perms:
                            a = a + _lane_gather(a, pidx)
                        wv = jnp.exp(a)
                        v0 = wv * xlj[2 * h]
                        v1 = wv * xlj[2 * h + 1]
                        ctb[e, pl.ds(32 * h, 16)] = jnp.where(dhi, z, v0)
                        ctb[e, pl.ds(32 * h + 16, 16)] = jnp.where(dhi, z, v1)
                        ctb[e, pl.ds(64 + 32 * h, 16)] = jnp.where(dhi, v0, z)
                        ctb[e, pl.ds(64 + 32 * h + 16, 16)] = jnp.where(
                            dhi, v1, z)
                        dv = dv + jnp.where(lane == h, wv, F32(0.0))
                    w = de * 4 + (2 * q + lane2)
                    plsc.addupdate_scatter(
                        denb, [lax.shift_right_logical(w, 7), w & 127], dv,
                        mask=mask2)

        issue_sd(0, 0)
        issue_d0sp(0, 0)

        @pl.loop(0, NCHUNK // 2)
        def _chunk(i2):
            for par in (0, 1):
                i = 2 * i2 + par
                other = 1 - par
                wait_sd(par)
                pltpu.async_copy(xl2.at[idx_s[par]], xlb, sem_g1)
                pltpu.async_copy(xr2.at[idx_d[par]], xrb, sem_g2)
                pltpu.async_copy(ee4.at[pl.ds(p * E + base_of(i), K)], eeb,
                                 sem_ee)
                if par == 0:
                    @pl.when(i2 > 0)
                    def _():
                        drain_sc()
                else:
                    drain_sc()
                issue_sd(i + 1, other)
                issue_d0sp(i + 1, other)
                wait_d0sp(par)
                pltpu.make_async_copy(ee4.at[pl.ds(0, K)], eeb, sem_ee).wait()
                pltpu.make_async_copy(xl2.at[idx_s[par]], xlb, sem_g1).wait()
                pltpu.make_async_copy(xr2.at[idx_d[par]], xrb, sem_g2).wait()
                compute(par)
                pltpu.async_copy(ctb, accum.at[idx_p], sem_sc, add=True)

        drain_sc()
        wait_sd(0)
        wait_d0sp(0)

        plsc.subcore_barrier()
        for kk in range(nz):
            @pl.when(jnp.logical_or(s < 15, kk < nz15))
            def _():
                r0 = s * _TR + kk * ZR
                pltpu.sync_copy(accum.at[pl.ds(r0, ZR)], zb)
                pltpu.sync_copy(zb, msg_out.at[pl.ds(p * _AR + r0, ZR)])
        if q == 0:
            @pl.loop(0, ZR)
            def _rezero(r):
                for j in range(DH // 16):
                    zb[r, pl.ds(j * 16, 16)] = jnp.zeros((16,), F32)
        plsc.subcore_barrier()

    pltpu.sync_copy(
        denb, den_out.at[pl.ds((c * NSUB + s) * (_DW // DH), _DW // DH)])


@functools.cache
def _edge_call():
    return pl.kernel(
        _edge_body,
        out_type=[jax.ShapeDtypeStruct((4 * _AR, DH), F32),
                  jax.ShapeDtypeStruct((2 * NSUB * (_DW // DH), DH), F32)],
        mesh=_mesh(),
        compiler_params=pltpu.CompilerParams(
            needs_layout_passes=False, internal_scratch_in_bytes=0),
        scratch_types=[
            [pltpu.VMEM((K,), jnp.int32)] * 2,
            [pltpu.VMEM((K,), jnp.int32)] * 2,
            [pltpu.VMEM((K,), jnp.int32)] * 2,
            pltpu.VMEM((K,), jnp.int32),
            pltpu.VMEM((K, DH), F32),
            pltpu.VMEM((K, DH), F32),
            pltpu.VMEM((K, 64), F32),
            pltpu.VMEM((K, DH), F32),
            pltpu.VMEM((ZR, DH), F32),
            pltpu.VMEM((DH,), F32),
            pltpu.VMEM((_DW // DH, DH), F32),
            pltpu.VMEM_SHARED((_AR, DH), F32),
            [pltpu.SemaphoreType.DMA] * 2,
            [pltpu.SemaphoreType.DMA] * 2,
            [pltpu.SemaphoreType.DMA] * 2,
            pltpu.SemaphoreType.DMA,
            pltpu.SemaphoreType.DMA,
            pltpu.SemaphoreType.DMA,
            pltpu.SemaphoreType.DMA,
        ],
    )


def _ln(x, g, b):
    m = jnp.mean(x, -1, keepdims=True)
    v = jnp.var(x, -1, keepdims=True)
    return (x - m) / jnp.sqrt(v + 1e-5) * g + b


_BN = 2000
_BE = 4000


def _fullb(shape):
    return pl.BlockSpec(shape, lambda i: (0,) * len(shape))


def _enc_body(x_ref, w1, b1, g1r, bl1, w2, b2, o_ref):
    h = jnp.dot(x_ref[...], w1[...], preferred_element_type=F32) + b1[...]
    h = jnp.maximum(h, 0.0)
    h = _ln(h, g1r[...], bl1[...])
    o_ref[...] = jnp.dot(h, w2[...], preferred_element_type=F32) + b2[...]


@functools.cache
def _enc_call():
    return pl.pallas_call(
        _enc_body,
        grid=(N // _BN,),
        in_specs=[pl.BlockSpec((_BN, 128), lambda i: (i, 0)),
                  _fullb((128, 256)), _fullb((256,)), _fullb((256,)),
                  _fullb((256,)), _fullb((256, 128)), _fullb((128,))],
        out_specs=pl.BlockSpec((_BN, 128), lambda i: (i, 0)),
        out_shape=jax.ShapeDtypeStruct((N, 128), F32),
    )


def _pack_halves(x, o_ref):
    for cc in range(2):
        o_ref[cc, :, :] = x[:, 128 * cc:128 * cc + 128]


def _pre_body(h_ref, la_ref, wl_r, blr, wr_r, brr, we_r, att_r, wres_r, bcr,
              xl4_ref, xr4_ref, res_ref, snum_ref, wl_ref):
    h = h_ref[...]
    xl = jnp.dot(h, wl_r[...], preferred_element_type=F32) + blr[...]
    xr = jnp.dot(h, wr_r[...], preferred_element_type=F32) + brr[...]
    res_ref[...] = (jnp.dot(h, wres_r[...], preferred_element_type=F32)
                    + bcr[...])
    _pack_halves(xl, xl4_ref)
    _pack_halves(xr, xr4_ref)
    ml = xl + xr + jnp.dot(la_ref[...], we_r[...], preferred_element_type=F32)
    ml = jnp.where(ml > 0, ml, 0.2 * ml)
    al = jnp.sum(ml.reshape(_BN, 8, 32) * att_r[...][None], -1)
    wl_ = jnp.exp(al)
    wl_ref[...] = wl_
    snum_ref[...] = jnp.repeat(wl_, 32, axis=1) * xl


@functools.cache
def _pre_call(din):
    return pl.pallas_call(
        _pre_body,
        grid=(N // _BN,),
        in_specs=[pl.BlockSpec((_BN, din), lambda i: (i, 0)),
                  pl.BlockSpec((_BN, 8), lambda i: (i, 0)),
                  _fullb((din, 256)), _fullb((256,)),
                  _fullb((din, 256)), _fullb((256,)),
                  _fullb((8, 256)), _fullb((8, 32)),
                  _fullb((din, 256)), _fullb((256,))],
        out_specs=[pl.BlockSpec((2, _BN, 128), lambda i: (0, i, 0)),
                   pl.BlockSpec((2, _BN, 128), lambda i: (0, i, 0)),
                   pl.BlockSpec((_BN, 256), lambda i: (i, 0)),
                   pl.BlockSpec((_BN, 256), lambda i: (i, 0)),
                   pl.BlockSpec((_BN, 8), lambda i: (i, 0))],
        out_shape=[jax.ShapeDtypeStruct((2, N, 128), F32),
                   jax.ShapeDtypeStruct((2, N, 128), F32),
                   jax.ShapeDtypeStruct((N, 256), F32),
                   jax.ShapeDtypeStruct((N, 256), F32),
                   jax.ShapeDtypeStruct((N, 8), F32)],
    )


def _ee_body(ea_ref, we_r, o_ref):
    ee = jnp.dot(ea_ref[...], we_r[...], preferred_element_type=F32)
    for p in range(4):
        o_ref[p, :, :] = ee[:, 64 * p:64 * p + 64]


@functools.cache
def _ee_call():
    return pl.pallas_call(
        _ee_body,
        grid=(E // _BE,),
        in_specs=[pl.BlockSpec((_BE, 8), lambda i: (i, 0)),
                  _fullb((8, 256))],
        out_specs=pl.BlockSpec((4, _BE, 64), lambda i: (0, i, 0)),
        out_shape=jax.ShapeDtypeStruct((4, E, 64), F32),
    )


def _post_body(m4_ref, dp_ref, snum_ref, wl_ref, res_ref, g_r, b_r, o_ref):
    m4 = m4_ref[...]
    parts = ([m4[p, :, 0:64] for p in range(4)]
             + [m4[p, :, 64:128] for p in range(4)])
    msg = jnp.concatenate(parts, axis=-1).reshape(_BN, 256)
    den8 = dp_ref[...] + wl_ref[...]
    num = msg + snum_ref[...]
    out = num / jnp.repeat(den8 + 1e-16, 32, axis=1) + res_ref[...]
    out = _ln(out, g_r[...], b_r[...])
    o_ref[...] = out * jax.nn.sigmoid(out)


@functools.cache
def _post_call():
    return pl.pallas_call(
        _post_body,
        grid=(N // _BN,),
        in_specs=[pl.BlockSpec((4, _BN // 2, 128), lambda i: (0, i, 0)),
                  pl.BlockSpec((_BN, 8), lambda i: (i, 0)),
                  pl.BlockSpec((_BN, 256), lambda i: (i, 0)),
                  pl.BlockSpec((_BN, 8), lambda i: (i, 0)),
                  pl.BlockSpec((_BN, 256), lambda i: (i, 0)),
                  _fullb((256,)), _fullb((256,))],
        out_specs=pl.BlockSpec((_BN, 256), lambda i: (i, 0)),
        out_shape=jax.ShapeDtypeStruct((N, 256), F32),
    )


def _pool_body(h_ref, idx_ref, wq, bq_r, wk, bk_r, wv, bv_r, wo, bo_r,
               o_ref, s_acc, c_acc):
    i = pl.program_id(0)

    @pl.when(i == 0)
    def _():
        s_acc[...] = jnp.zeros_like(s_acc)
        c_acc[...] = jnp.zeros_like(c_acc)

    seg = jax.lax.broadcasted_iota(jnp.int32, (160, _BN), 0)
    oh = (seg == idx_ref[...][0]).astype(F32)
    s_acc[...] += jnp.dot(oh, h_ref[...], preferred_element_type=F32)
    c_acc[...] += jnp.sum(oh, axis=1, keepdims=True)

    @pl.when(i == N // _BN - 1)
    def _():
        dense = s_acc[...] / jnp.maximum(c_acc[...], 1.0)
        q0 = dense.reshape(40, 4, 256)[:, 0, :]
        q = jnp.dot(q0, wq[...], preferred_element_type=F32) + bq_r[...]
        kk = jnp.dot(dense, wk[...], preferred_element_type=F32) + bk_r[...]
        vv = jnp.dot(dense, wv[...], preferred_element_type=F32) + bv_r[...]
        q4 = q.reshape(40, 1, 8, 32)
        k4 = kk.reshape(40, 4, 8, 32)
        v4 = vv.reshape(40, 4, 8, 32)
        logits = jnp.sum(q4 * k4, -1) / jnp.sqrt(F32(32.0))
        mxl = jnp.max(logits, axis=1, keepdims=True)
        ex = jnp.exp(logits - mxl)
        aw = ex / jnp.sum(ex, axis=1, keepdims=True)
        o = jnp.sum(aw[..., None] * v4, axis=1).reshape(40, 256)
        o_ref[...] = jnp.dot(o, wo[...], preferred_element_type=F32) + bo_r[...]


@functools.cache
def _pool_call():
    return pl.pallas_call(
        _pool_body,
        grid=(N // _BN,),
        in_specs=[pl.BlockSpec((_BN, 256), lambda i: (i, 0)),
                  pl.BlockSpec((1, 1, _BN), lambda i: (i, 0, 0)),
                  _fullb((256, 256)), _fullb((256,)),
                  _fullb((256, 256)), _fullb((256,)),
                  _fullb((256, 256)), _fullb((256,)),
                  _fullb((256, 256)), _fullb((256,))],
        out_specs=pl.BlockSpec((40, 256), lambda i: (0, 0)),
        out_shape=jax.ShapeDtypeStruct((40, 256), F32),
        scratch_shapes=[pltpu.VMEM((160, 256), F32),
                        pltpu.VMEM((160, 1), F32)],
    )


def kernel(x, edge_index, edge_attr, batch, t, node_type, node_temporal_mask,
           enc_W1, enc_b1, enc_g, enc_bln, enc_W2, enc_b2,
           Wl0, bl0, Wr0, br0, We0, att0, Wres0, bc0, g0, bn0,
           Wl1, bl1, Wr1, br1, We1, att1, Wres1, bc1, g1, bn1,
           Wl2, bl2, Wr2, br2, We2, att2, Wres2, bc2, g2, bn2,
           Wq, Wk, Wv, Wo, bq, bk, bv, bo):
    src = edge_index[0].astype(jnp.int32)
    dst = edge_index[1].astype(jnp.int32)
    off2 = jnp.repeat(jnp.arange(2, dtype=jnp.int32) * N, E)
    src2 = jnp.tile(src, 2) + off2
    dst2 = jnp.tile(dst, 2) + off2

    eapf = jnp.concatenate(
        [edge_attr, jnp.ones((E, 1), F32), jnp.zeros((E, 1), F32)],
        axis=1).reshape(-1)
    lsum2 = _loopattr_call()(eapf, dst)
    lsum = lsum2.reshape(2 * NSUB, NP, 8).sum(axis=0)[:N]
    cnt = jnp.maximum(lsum[:, 6], 1.0)
    la8 = jnp.concatenate(
        [lsum[:, :6] / cnt[:, None], jnp.zeros((N, 2), F32)], axis=1)

    h = _enc_call()(x, enc_W1, enc_b1, enc_g, enc_bln, enc_W2, enc_b2)
    eap8 = jnp.pad(edge_attr, ((0, 0), (0, 2)))

    layers = [
        (Wl0, bl0, Wr0, br0, We0, att0, Wres0, bc0, g0, bn0),
        (Wl1, bl1, Wr1, br1, We1, att1, Wres1, bc1, g1, bn1),
        (Wl2, bl2, Wr2, br2, We2, att2, Wres2, bc2, g2, bn2),
    ]
    for (Wl, bl, Wr, br, We, att, Wres, bc, g, bn) in layers:
        din = Wl.shape[0]
        We8 = jnp.pad(We, ((0, 2), (0, 0)))
        xl4, xr4, res, snum, wl_ = _pre_call(din)(
            h, la8, Wl, bl, Wr, br, We8, att, Wres, bc)
        ee4 = _ee_call()(eap8, We8).reshape(4 * E, 64)
        att4 = jnp.pad(att.reshape(4, 64), ((0, 0), (0, 64))).reshape(-1)
        msg4, denp = _edge_call()(
            src2, dst, dst2, ee4, xl4.reshape(2 * N, 128),
            xr4.reshape(2 * N, 128), att4)
        m4 = msg4.reshape(4, _AR, 128)[:, :N // 2]
        denh = denp.reshape(2, NSUB, NP, 4).sum(axis=1)
        den8 = jnp.concatenate([denh[0, :N], denh[1, :N]], axis=1)
        h = _post_call()(m4, den8, snum, wl_, res, g, bn)

    Tv = node_temporal_mask.shape[1]
    Bv = node_type.shape[0]
    ntf = jnp.broadcast_to(node_type[:, None, :],
                           (Bv, Tv, node_type.shape[1])).reshape(-1)
    tmf = node_temporal_mask.reshape(-1)
    pool_idx = (batch * (Tv * 4) + tmf * 4 + ntf).astype(jnp.int32)
    o = _pool_call()(h, pool_idx.reshape(N // _BN, 1, _BN),
                     Wq, bq, Wk, bk, Wv, bv, Wo, bo)
    return o.reshape(Bv, Tv, HID)
